# baseline (speedup 1.0000x reference)
import jax
import jax.numpy as jnp
from jax import lax
from jax.experimental import pallas as pl
from jax.experimental.pallas import tpu as pltpu
from jax.experimental.pallas import tpu_sc as plsc

N_NODES = 99999
N_EDGES = N_NODES * 64
N_OUT = N_NODES // 3
ACC = 33792
X_PAD = 100000

NC, NS = 2, 16
NW = NC * NS
BLK = 4096
NB_FULL = N_EDGES // BLK
TAIL = N_EDGES - NB_FULL * BLK
NB_EXTRA = NB_FULL - (NB_FULL // NW) * NW
NPAIR = (NB_FULL // NW) // 2
ZSEG = ACC // NS


def _sc_scatter(x_hbm, ei_hbm, p_hbm,
                x_v, src_v, dst_v, idx_v, val_v, idx_w, val_w,
                idx_t, val_t, seg_v, acc_sh, sem, sem2):
    cid = lax.axis_index("c")
    sid = lax.axis_index("s")
    wid = sid * NC + cid

    pltpu.sync_copy(x_hbm, x_v)

    def zstep(j, c):
        seg_v[pl.ds(j * 16, 16)] = jnp.zeros((16,), jnp.float32)
        return c
    lax.fori_loop(0, ZSEG // 16, zstep, 0)
    pltpu.sync_copy(seg_v, acc_sh.at[pl.ds(sid * ZSEG, ZSEG)])
    plsc.subcore_barrier()

    three = jnp.int32(3)
    third = jnp.float32(1.0 / 3.0)

    def step(sref, dref, iref, vref, o, oo):
        s16 = sref[pl.ds(o, 16)]
        d16 = dref[pl.ds(o, 16)]
        xs = plsc.load_gather(x_v, [s16])
        q = (d16.astype(jnp.float32) * third).astype(jnp.int32)
        r = d16 - q * three
        iref[pl.ds(oo, 16)] = q
        vref[pl.ds(oo, 16)] = jnp.where(r == 0, xs, jnp.float32(0.0))

    def fill(boff, idx_b, val_b):
        pltpu.sync_copy(ei_hbm.at[pl.ds(boff, BLK)], src_v)
        pltpu.sync_copy(ei_hbm.at[pl.ds(N_EDGES + boff, BLK)], dst_v)

        def group(t, c2):
            for j in range(8):
                o = t * 128 + j * 16
                step(src_v, dst_v, idx_b, val_b, o, o)
            return c2
        lax.fori_loop(0, BLK // 128, group, 0)

    def pair(h, c):
        boff0 = (2 * h * NW + wid) * BLK

        @pl.when(h > 0)
        def _():
            pltpu.make_async_copy(val_v, acc_sh.at[idx_v], sem).wait()
        fill(boff0, idx_v, val_v)
        pltpu.async_copy(val_v, acc_sh.at[idx_v], sem, add=True)

        @pl.when(h > 0)
        def _():
            pltpu.make_async_copy(val_w, acc_sh.at[idx_w], sem2).wait()
        fill(boff0 + NW * BLK, idx_w, val_w)
        pltpu.async_copy(val_w, acc_sh.at[idx_w], sem2, add=True)
        return c
    lax.fori_loop(0, NPAIR, pair, 0)
    pltpu.make_async_copy(val_v, acc_sh.at[idx_v], sem).wait()
    pltpu.make_async_copy(val_w, acc_sh.at[idx_w], sem2).wait()

    @pl.when(wid < NB_EXTRA)
    def _():
        fill((2 * NPAIR * NW + wid) * BLK, idx_v, val_v)
        pltpu.sync_copy(val_v, acc_sh.at[idx_v], add=True)

    @pl.when(wid == NW - 1)
    def _():
        toff = NB_FULL * BLK
        pltpu.sync_copy(ei_hbm.at[pl.ds(toff, TAIL)],
                        src_v.at[pl.ds(0, TAIL)])
        pltpu.sync_copy(ei_hbm.at[pl.ds(N_EDGES + toff, TAIL)],
                        dst_v.at[pl.ds(0, TAIL)])

        for cpart in range(2):
            def tstep(j, c, _cp=cpart):
                step(src_v, dst_v, idx_t, val_t,
                     _cp * (TAIL // 2) + j * 16, j * 16)
                return c
            lax.fori_loop(0, TAIL // 32, tstep, 0)
            pltpu.sync_copy(val_t, acc_sh.at[idx_t], add=True)

    plsc.subcore_barrier()

    pltpu.sync_copy(acc_sh.at[pl.ds(sid * ZSEG, ZSEG)], seg_v)
    pltpu.sync_copy(seg_v, p_hbm.at[pl.ds(cid * ACC + sid * ZSEG, ZSEG)])


def _combine(consts_ref, p0_ref, p1_ref, x3_ref, out_ref):
    wl = consts_ref[0]
    wr = consts_ref[1]
    w10 = consts_ref[2]
    w11 = consts_ref[3]
    b10 = consts_ref[4]
    b11 = consts_ref[5]
    w20 = consts_ref[6]
    w21 = consts_ref[7]
    b2v = consts_ref[8]
    h = wl * (p0_ref[...] + p1_ref[...]) + wr * x3_ref[...]
    a0 = jnp.maximum(w10 * h + b10, 0.0)
    a1 = jnp.maximum(w11 * h + b11, 0.0)
    out_ref[...] = w20 * a0 + w21 * a1 + b2v


def kernel(x, edge_index, W_l, W_r, w1, b1, w2, b2):
    xf = x.reshape(-1)
    x_p = jnp.concatenate(
        [xf, jnp.zeros((X_PAD - N_NODES,), jnp.float32)])
    ei_flat = edge_index.reshape(-1)

    mesh = plsc.VectorSubcoreMesh(core_axis_name="c", subcore_axis_name="s")
    partials = pl.kernel(
        _sc_scatter,
        out_type=jax.ShapeDtypeStruct((NC * ACC,), jnp.float32),
        mesh=mesh,
        compiler_params=pltpu.CompilerParams(needs_layout_passes=False),
        scratch_types=[
            pltpu.VMEM((X_PAD,), jnp.float32),
            pltpu.VMEM((BLK,), jnp.int32),
            pltpu.VMEM((BLK,), jnp.int32),
            pltpu.VMEM((BLK,), jnp.int32),
            pltpu.VMEM((BLK,), jnp.float32),
            pltpu.VMEM((BLK,), jnp.int32),
            pltpu.VMEM((BLK,), jnp.float32),
            pltpu.VMEM((TAIL // 2,), jnp.int32),
            pltpu.VMEM((TAIL // 2,), jnp.float32),
            pltpu.VMEM((ZSEG,), jnp.float32),
            pltpu.VMEM_SHARED((ACC,), jnp.float32),
            pltpu.SemaphoreType.DMA,
            pltpu.SemaphoreType.DMA,
        ],
    )(x_p, ei_flat)

    x3 = xf[: N_OUT * 3].reshape(N_OUT, 3)[:, 0]
    x3_p = jnp.concatenate(
        [x3, jnp.zeros((ACC - N_OUT,), jnp.float32)]).reshape(264, 128)
    consts = jnp.concatenate([
        W_l.reshape(-1), W_r.reshape(-1), w1.reshape(-1),
        b1.reshape(-1), w2.reshape(-1), b2.reshape(-1),
        jnp.zeros((7,), jnp.float32),
    ])

    out2d = pl.pallas_call(
        _combine,
        out_shape=jax.ShapeDtypeStruct((264, 128), jnp.float32),
        in_specs=[
            pl.BlockSpec(memory_space=pltpu.SMEM),
            pl.BlockSpec(memory_space=pltpu.VMEM),
            pl.BlockSpec(memory_space=pltpu.VMEM),
            pl.BlockSpec(memory_space=pltpu.VMEM),
        ],
        out_specs=pl.BlockSpec(memory_space=pltpu.VMEM),
    )(consts, partials[:ACC].reshape(264, 128),
      partials[ACC:].reshape(264, 128), x3_p)

    return out2d.reshape(-1)[:N_OUT]


# speedup vs baseline: 4.7954x; 4.7954x over previous
import jax
import jax.numpy as jnp
from jax import lax
from jax.experimental import pallas as pl
from jax.experimental.pallas import tpu as pltpu
from jax.experimental.pallas import tpu_sc as plsc

N_NODES = 99999
N_EDGES = N_NODES * 64
N_OUT = N_NODES // 3
ACC = 33792
X_PAD = 100000

NC, NS = 2, 16
NW = NC * NS
BLK = 4096
NB_FULL = N_EDGES // BLK
TAIL = N_EDGES - NB_FULL * BLK
NB_EXTRA = NB_FULL - (NB_FULL // NW) * NW
NPAIR = (NB_FULL // NW) // 2
ZSEG = ACC // NS


def _sc_scatter(x_hbm, src_hbm, dst_hbm, p_hbm,
                x_v, src_v, dst_v, idx_v, val_v, idx_w, val_w,
                idx_t, val_t, seg_v, acc_sh, sem, sem2):
    cid = lax.axis_index("c")
    sid = lax.axis_index("s")
    wid = sid * NC + cid

    pltpu.sync_copy(x_hbm, x_v)

    def zstep(j, c):
        seg_v[pl.ds(j * 16, 16)] = jnp.zeros((16,), jnp.float32)
        return c
    lax.fori_loop(0, ZSEG // 16, zstep, 0)
    pltpu.sync_copy(seg_v, acc_sh.at[pl.ds(sid * ZSEG, ZSEG)])
    plsc.subcore_barrier()

    three = jnp.int32(3)
    third = jnp.float32(1.0 / 3.0)

    def step(sref, dref, iref, vref, o, oo):
        s16 = sref[pl.ds(o, 16)]
        d16 = dref[pl.ds(o, 16)]
        xs = plsc.load_gather(x_v, [s16])
        q = (d16.astype(jnp.float32) * third).astype(jnp.int32)
        r = d16 - q * three
        iref[pl.ds(oo, 16)] = q
        vref[pl.ds(oo, 16)] = jnp.where(r == 0, xs, jnp.float32(0.0))

    def fill(boff, idx_b, val_b):
        pltpu.sync_copy(src_hbm.at[pl.ds(boff, BLK)], src_v)
        pltpu.sync_copy(dst_hbm.at[pl.ds(boff, BLK)], dst_v)

        def group(t, c2):
            for j in range(8):
                o = t * 128 + j * 16
                step(src_v, dst_v, idx_b, val_b, o, o)
            return c2
        lax.fori_loop(0, BLK // 128, group, 0)

    def pair(h, c):
        boff0 = (2 * h * NW + wid) * BLK

        @pl.when(h > 0)
        def _():
            pltpu.make_async_copy(val_v, acc_sh.at[idx_v], sem).wait()
        fill(boff0, idx_v, val_v)
        pltpu.async_copy(val_v, acc_sh.at[idx_v], sem, add=True)

        @pl.when(h > 0)
        def _():
            pltpu.make_async_copy(val_w, acc_sh.at[idx_w], sem2).wait()
        fill(boff0 + NW * BLK, idx_w, val_w)
        pltpu.async_copy(val_w, acc_sh.at[idx_w], sem2, add=True)
        return c
    lax.fori_loop(0, NPAIR, pair, 0)
    pltpu.make_async_copy(val_v, acc_sh.at[idx_v], sem).wait()
    pltpu.make_async_copy(val_w, acc_sh.at[idx_w], sem2).wait()

    @pl.when(wid < NB_EXTRA)
    def _():
        fill((2 * NPAIR * NW + wid) * BLK, idx_v, val_v)
        pltpu.sync_copy(val_v, acc_sh.at[idx_v], add=True)

    @pl.when(wid == NW - 1)
    def _():
        toff = NB_FULL * BLK
        pltpu.sync_copy(src_hbm.at[pl.ds(toff, TAIL)],
                        src_v.at[pl.ds(0, TAIL)])
        pltpu.sync_copy(dst_hbm.at[pl.ds(toff, TAIL)],
                        dst_v.at[pl.ds(0, TAIL)])

        for cpart in range(2):
            def tstep(j, c, _cp=cpart):
                step(src_v, dst_v, idx_t, val_t,
                     _cp * (TAIL // 2) + j * 16, j * 16)
                return c
            lax.fori_loop(0, TAIL // 32, tstep, 0)
            pltpu.sync_copy(val_t, acc_sh.at[idx_t], add=True)

    plsc.subcore_barrier()

    pltpu.sync_copy(acc_sh.at[pl.ds(sid * ZSEG, ZSEG)], seg_v)
    pltpu.sync_copy(seg_v, p_hbm.at[pl.ds(cid * ACC + sid * ZSEG, ZSEG)])


def _combine(consts_ref, p0_ref, p1_ref, x3_ref, out_ref):
    wl = consts_ref[0]
    wr = consts_ref[1]
    w10 = consts_ref[2]
    w11 = consts_ref[3]
    b10 = consts_ref[4]
    b11 = consts_ref[5]
    w20 = consts_ref[6]
    w21 = consts_ref[7]
    b2v = consts_ref[8]
    h = wl * (p0_ref[...] + p1_ref[...]) + wr * x3_ref[...]
    a0 = jnp.maximum(w10 * h + b10, 0.0)
    a1 = jnp.maximum(w11 * h + b11, 0.0)
    out_ref[...] = w20 * a0 + w21 * a1 + b2v


def kernel(x, edge_index, W_l, W_r, w1, b1, w2, b2):
    xf = x.reshape(-1)
    x_p = jnp.concatenate(
        [xf, jnp.zeros((X_PAD - N_NODES,), jnp.float32)])
    src = edge_index[0]
    dst = edge_index[1]

    mesh = plsc.VectorSubcoreMesh(core_axis_name="c", subcore_axis_name="s")
    partials = pl.kernel(
        _sc_scatter,
        out_type=jax.ShapeDtypeStruct((NC * ACC,), jnp.float32),
        mesh=mesh,
        compiler_params=pltpu.CompilerParams(needs_layout_passes=False),
        scratch_types=[
            pltpu.VMEM((X_PAD,), jnp.float32),
            pltpu.VMEM((BLK,), jnp.int32),
            pltpu.VMEM((BLK,), jnp.int32),
            pltpu.VMEM((BLK,), jnp.int32),
            pltpu.VMEM((BLK,), jnp.float32),
            pltpu.VMEM((BLK,), jnp.int32),
            pltpu.VMEM((BLK,), jnp.float32),
            pltpu.VMEM((TAIL // 2,), jnp.int32),
            pltpu.VMEM((TAIL // 2,), jnp.float32),
            pltpu.VMEM((ZSEG,), jnp.float32),
            pltpu.VMEM_SHARED((ACC,), jnp.float32),
            pltpu.SemaphoreType.DMA,
            pltpu.SemaphoreType.DMA,
        ],
    )(x_p, src, dst)

    x3 = xf[: N_OUT * 3].reshape(N_OUT, 3)[:, 0]
    x3_p = jnp.concatenate(
        [x3, jnp.zeros((ACC - N_OUT,), jnp.float32)]).reshape(264, 128)
    consts = jnp.concatenate([
        W_l.reshape(-1), W_r.reshape(-1), w1.reshape(-1),
        b1.reshape(-1), w2.reshape(-1), b2.reshape(-1),
        jnp.zeros((7,), jnp.float32),
    ])

    out2d = pl.pallas_call(
        _combine,
        out_shape=jax.ShapeDtypeStruct((264, 128), jnp.float32),
        in_specs=[
            pl.BlockSpec(memory_space=pltpu.SMEM),
            pl.BlockSpec(memory_space=pltpu.VMEM),
            pl.BlockSpec(memory_space=pltpu.VMEM),
            pl.BlockSpec(memory_space=pltpu.VMEM),
        ],
        out_specs=pl.BlockSpec(memory_space=pltpu.VMEM),
    )(consts, partials[:ACC].reshape(264, 128),
      partials[ACC:].reshape(264, 128), x3_p)

    return out2d.reshape(-1)[:N_OUT]


# speedup vs baseline: 5.1923x; 1.0828x over previous
import jax
import jax.numpy as jnp
from jax import lax
from jax.experimental import pallas as pl
from jax.experimental.pallas import tpu as pltpu
from jax.experimental.pallas import tpu_sc as plsc

N_NODES = 99999
N_EDGES = N_NODES * 64
N_OUT = N_NODES // 3
ACC = 33792
X_PAD = 100000

NC, NS = 2, 16
NW = NC * NS
BLK = 4096
NB_FULL = N_EDGES // BLK
REM = N_EDGES - NB_FULL * BLK
TAIL = (REM // 128) * 128
NREM = REM - TAIL
NB_EXTRA = NB_FULL - (NB_FULL // NW) * NW
NPAIR = (NB_FULL // NW) // 2
ZSEG = ACC // NS


def _sc_scatter(x_hbm, ei_hbm, p_hbm,
                x_v, ed_v, idx_v, val_v, idx_w, val_w,
                idx_t, val_t, seg_v, acc_sh, sem, sem2):
    cid = lax.axis_index("c")
    sid = lax.axis_index("s")
    wid = sid * NC + cid

    pltpu.sync_copy(x_hbm, x_v)

    def zstep(j, c):
        seg_v[pl.ds(j * 16, 16)] = jnp.zeros((16,), jnp.float32)
        return c
    lax.fori_loop(0, ZSEG // 16, zstep, 0)
    pltpu.sync_copy(seg_v, acc_sh.at[pl.ds(sid * ZSEG, ZSEG)])
    plsc.subcore_barrier()

    three = jnp.int32(3)
    third = jnp.float32(1.0 / 3.0)

    def step(iref, vref, o, oo):
        s16 = ed_v[0, pl.ds(o, 16)]
        d16 = ed_v[1, pl.ds(o, 16)]
        xs = plsc.load_gather(x_v, [s16])
        q = (d16.astype(jnp.float32) * third).astype(jnp.int32)
        r = d16 - q * three
        iref[pl.ds(oo, 16)] = q
        vref[pl.ds(oo, 16)] = jnp.where(r == 0, xs, jnp.float32(0.0))

    def fill(boff, idx_b, val_b):
        pltpu.sync_copy(ei_hbm.at[:, pl.ds(boff, BLK)], ed_v)

        def group(t, c2):
            for j in range(8):
                o = t * 128 + j * 16
                step(idx_b, val_b, o, o)
            return c2
        lax.fori_loop(0, BLK // 128, group, 0)

    def pair(h, c):
        boff0 = (2 * h * NW + wid) * BLK

        @pl.when(h > 0)
        def _():
            pltpu.make_async_copy(val_v, acc_sh.at[idx_v], sem).wait()
        fill(boff0, idx_v, val_v)
        pltpu.async_copy(val_v, acc_sh.at[idx_v], sem, add=True)

        @pl.when(h > 0)
        def _():
            pltpu.make_async_copy(val_w, acc_sh.at[idx_w], sem2).wait()
        fill(boff0 + NW * BLK, idx_w, val_w)
        pltpu.async_copy(val_w, acc_sh.at[idx_w], sem2, add=True)
        return c
    lax.fori_loop(0, NPAIR, pair, 0)
    pltpu.make_async_copy(val_v, acc_sh.at[idx_v], sem).wait()
    pltpu.make_async_copy(val_w, acc_sh.at[idx_w], sem2).wait()

    @pl.when(wid < NB_EXTRA)
    def _():
        fill((2 * NPAIR * NW + wid) * BLK, idx_v, val_v)
        pltpu.sync_copy(val_v, acc_sh.at[idx_v], add=True)

    @pl.when(wid == NW - 1)
    def _():
        pltpu.sync_copy(ei_hbm.at[:, pl.ds(NB_FULL * BLK, TAIL)],
                        ed_v.at[:, pl.ds(0, TAIL)])
        for cpart in range(2):
            def tstep(j, c, _cp=cpart):
                step(idx_t, val_t, _cp * (TAIL // 2) + j * 16, j * 16)
                return c
            lax.fori_loop(0, TAIL // 32, tstep, 0)
            pltpu.sync_copy(val_t, acc_sh.at[idx_t], add=True)

    plsc.subcore_barrier()

    pltpu.sync_copy(acc_sh.at[pl.ds(sid * ZSEG, ZSEG)], seg_v)
    pltpu.sync_copy(seg_v, p_hbm.at[pl.ds(cid * ACC + sid * ZSEG, ZSEG)])


def _combine(consts_ref, q_ref, v_ref, p0_ref, p1_ref, x3_ref, out_ref):
    wl = consts_ref[0]
    wr = consts_ref[1]
    w10 = consts_ref[2]
    w11 = consts_ref[3]
    b10 = consts_ref[4]
    b11 = consts_ref[5]
    w20 = consts_ref[6]
    w21 = consts_ref[7]
    b2v = consts_ref[8]

    lin = (lax.broadcasted_iota(jnp.int32, (264, 128), 0) * 128
           + lax.broadcasted_iota(jnp.int32, (264, 128), 1))

    def estep(e, acc):
        return acc + jnp.where(lin == q_ref[e], v_ref[e], 0.0)
    tail_acc = lax.fori_loop(
        0, NREM, estep, jnp.zeros((264, 128), jnp.float32))

    h = wl * (p0_ref[...] + p1_ref[...] + tail_acc) + wr * x3_ref[...]
    a0 = jnp.maximum(w10 * h + b10, 0.0)
    a1 = jnp.maximum(w11 * h + b11, 0.0)
    out_ref[...] = w20 * a0 + w21 * a1 + b2v


def kernel(x, edge_index, W_l, W_r, w1, b1, w2, b2):
    xf = x.reshape(-1)
    x_p = jnp.concatenate(
        [xf, jnp.zeros((X_PAD - N_NODES,), jnp.float32)])

    mesh = plsc.VectorSubcoreMesh(core_axis_name="c", subcore_axis_name="s")
    partials = pl.kernel(
        _sc_scatter,
        out_type=jax.ShapeDtypeStruct((NC * ACC,), jnp.float32),
        mesh=mesh,
        compiler_params=pltpu.CompilerParams(needs_layout_passes=False),
        scratch_types=[
            pltpu.VMEM((X_PAD,), jnp.float32),
            pltpu.VMEM((2, BLK), jnp.int32),
            pltpu.VMEM((BLK,), jnp.int32),
            pltpu.VMEM((BLK,), jnp.float32),
            pltpu.VMEM((BLK,), jnp.int32),
            pltpu.VMEM((BLK,), jnp.float32),
            pltpu.VMEM((TAIL // 2,), jnp.int32),
            pltpu.VMEM((TAIL // 2,), jnp.float32),
            pltpu.VMEM((ZSEG,), jnp.float32),
            pltpu.VMEM_SHARED((ACC,), jnp.float32),
            pltpu.SemaphoreType.DMA,
            pltpu.SemaphoreType.DMA,
        ],
    )(x_p, edge_index)

    src_r = edge_index[0, N_EDGES - NREM:]
    dst_r = edge_index[1, N_EDGES - NREM:]
    q_r = dst_r // 3
    v_r = jnp.where(dst_r % 3 == 0, xf[src_r], 0.0).astype(jnp.float32)

    x3 = xf[: N_OUT * 3].reshape(N_OUT, 3)[:, 0]
    x3_p = jnp.concatenate(
        [x3, jnp.zeros((ACC - N_OUT,), jnp.float32)]).reshape(264, 128)
    consts = jnp.concatenate([
        W_l.reshape(-1), W_r.reshape(-1), w1.reshape(-1),
        b1.reshape(-1), w2.reshape(-1), b2.reshape(-1),
        jnp.zeros((7,), jnp.float32),
    ])

    out2d = pl.pallas_call(
        _combine,
        out_shape=jax.ShapeDtypeStruct((264, 128), jnp.float32),
        in_specs=[
            pl.BlockSpec(memory_space=pltpu.SMEM),
            pl.BlockSpec(memory_space=pltpu.SMEM),
            pl.BlockSpec(memory_space=pltpu.SMEM),
            pl.BlockSpec(memory_space=pltpu.VMEM),
            pl.BlockSpec(memory_space=pltpu.VMEM),
            pl.BlockSpec(memory_space=pltpu.VMEM),
        ],
        out_specs=pl.BlockSpec(memory_space=pltpu.VMEM),
    )(consts, q_r, v_r, partials[:ACC].reshape(264, 128),
      partials[ACC:].reshape(264, 128), x3_p)

    return out2d.reshape(-1)[:N_OUT]
